# Initial kernel scaffold; baseline (speedup 1.0000x reference)
import jax
import jax.numpy as jnp
from jax.experimental import pallas as pl


def kernel(x):
    raise NotImplementedError("write your pallas kernel here")


# speedup vs baseline: 2.8679x; 2.8679x over previous
import jax
import jax.numpy as jnp
from jax import lax
from jax.experimental import pallas as pl
from jax.experimental.pallas import tpu as pltpu
from jax.experimental.pallas import tpu_sc as plsc
import functools

B = 16
N = 200000
H = 720
W = 1280
NC = 2
NS = 16
L = 16
BAND = H // NS
CH = 8000
NCHUNK = N // CH
GROUPS = CH // L
OROWS = 5
OCHUNK = BAND // OROWS
OWORDS = OROWS * W * 3
OGROUPS = OROWS * W // L


def _sc_body(x_hbm, out_hbm, band, evbuf, outbuf):
    c = lax.axis_index("c")
    s = lax.axis_index("s")
    row_lo = s * BAND

    lane = lax.iota(jnp.int32, L)
    lane4 = lane * 4
    lane2 = lane * 2
    lane3 = lane * 3
    neg1 = jnp.full((L,), -1, jnp.int32)

    def _init(i, _):
        band[pl.ds(i * L, L)] = neg1
        return 0
    lax.fori_loop(0, BAND * W // L, _init, 0)

    def _batch(bi, _):
        b = c * 8 + bi

        def _chunk(ci, _):
            pltpu.sync_copy(x_hbm.at[b, pl.ds(ci * (CH * 4), CH * 4)], evbuf)

            def _group(g, _):
                fbase = g * (L * 4)
                idxx = lane4 + (fbase + 1)
                xv = plsc.load_gather(evbuf, [idxx])
                yv = plsc.load_gather(evbuf, [idxx + 1])
                pv = plsc.load_gather(evbuf, [idxx + 2])
                row = yv.astype(jnp.int32)
                col = xv.astype(jnp.int32)
                m = (row >= row_lo) & (row < row_lo + BAND)
                pix = (row - row_lo) * W + col
                pix = jnp.where(m, pix, 0)
                kbase = ci * (CH * 2) + g * (L * 2)
                key = kbase + lane2 + pv.astype(jnp.int32)
                plsc.store_scatter(band, [pix], key, mask=m)
                back = plsc.load_gather(band, [pix], mask=m)
                need = m & (back < key)

                def _fix(n):
                    plsc.store_scatter(band, [pix], key, mask=n)
                    bk = plsc.load_gather(band, [pix], mask=n)
                    return n & (bk < key)
                lax.while_loop(lambda n: jnp.any(n), _fix, need)
                return 0
            lax.fori_loop(0, GROUPS, _group, 0)
            return 0
        lax.fori_loop(0, NCHUNK, _chunk, 0)

        def _ochunk(oc, _):
            def _pgroup(v, _):
                pbase = oc * (OROWS * W) + v * L
                key = band[pl.ds(pbase, L)]
                band[pl.ds(pbase, L)] = neg1
                touched = key >= 0
                polb = (key & 1) == 1
                y0 = jnp.where(touched & polb, 0.0, 255.0).astype(jnp.float32)
                y1 = jnp.where(touched & (~polb), 0.0, 255.0).astype(jnp.float32)
                y2 = y0 + y1
                oidx = lane3 + v * (L * 3)
                plsc.store_scatter(outbuf, [oidx], y0)
                plsc.store_scatter(outbuf, [oidx + 1], y1)
                plsc.store_scatter(outbuf, [oidx + 2], y2)
                return 0
            lax.fori_loop(0, OGROUPS, _pgroup, 0)
            off = (row_lo + oc * OROWS) * (W * 3)
            pltpu.sync_copy(outbuf, out_hbm.at[b, pl.ds(off, OWORDS)])
            return 0
        lax.fori_loop(0, OCHUNK, _ochunk, 0)
        return 0
    lax.fori_loop(0, B // NC, _batch, 0)


@jax.jit
def _e2img(xr):
    mesh = plsc.VectorSubcoreMesh(
        core_axis_name="c", subcore_axis_name="s", num_cores=NC, num_subcores=NS)
    f = pl.kernel(
        _sc_body,
        out_type=jax.ShapeDtypeStruct((B, H * W * 3), jnp.float32),
        mesh=mesh,
        scratch_types=[
            pltpu.VMEM((BAND * W,), jnp.int32),
            pltpu.VMEM((CH * 4,), jnp.float32),
            pltpu.VMEM((OWORDS,), jnp.float32),
        ],
        compiler_params=pltpu.CompilerParams(needs_layout_passes=False),
    )
    return f(xr)


def kernel(x):
    xr = x.reshape(B, N * 4)
    out = _e2img(xr)
    return out.reshape(B, H, W, 3)


# speedup vs baseline: 4.6098x; 1.6074x over previous
import jax
import jax.numpy as jnp
from jax import lax
from jax.experimental import pallas as pl
from jax.experimental.pallas import tpu as pltpu
from jax.experimental.pallas import tpu_sc as plsc

B = 16
N = 200000
H = 720
W = 1280
NC = 2
NS = 16
L = 16
BAND = H // NS
CH = 4000
CHW = CH * 4
NCHUNK = N // CH
GROUPS = CH // L
GUNROLL = 5
OROWS = 3
OCHUNK = BAND // OROWS
OWORDS = OROWS * W * 3
OGROUPS = OROWS * W // L
OUNROLL = 4


def _sc_body(x_hbm, out_hbm, band, ev0, ev1, ob0, ob1, se0, se1, so0, so1):
    c = lax.axis_index("c")
    s = lax.axis_index("s")
    row_lo = s * BAND

    lane = lax.iota(jnp.int32, L)
    lane4 = lane * 4
    lane2 = lane * 2
    lane3 = lane * 3
    neg1 = jnp.full((L,), -1, jnp.int32)
    falses = jnp.zeros((L,), jnp.bool_)

    def _init(i, _):
        base = i * (L * 8)
        for u in range(8):
            band[pl.ds(base + u * L, L)] = neg1
        return 0
    lax.fori_loop(0, BAND * W // (L * 8), _init, 0)

    def _ev_group(buf, ci, g, mode_key_only=False):
        fbase = g * (L * 4)
        idx = lane4 + (fbase + 1)
        xv = plsc.load_gather(buf, [idx])
        yv = plsc.load_gather(buf, [idx + 1])
        pv = plsc.load_gather(buf, [idx + 2])
        rl = yv.astype(jnp.int32) - row_lo
        col = xv.astype(jnp.int32)
        m = (rl >= 0) & (rl < BAND)
        pix = jnp.where(m, rl * W + col, 0)
        key = (ci * (CH * 2) + g * (L * 2)) + lane2 + pv.astype(jnp.int32)
        return m, pix, key

    def _batch(bi, _):
        b = c * 8 + bi

        pltpu.async_copy(x_hbm.at[b, pl.ds(0, CHW)], ev0, se0)
        pltpu.async_copy(x_hbm.at[b, pl.ds(CHW, CHW)], ev1, se1)

        def _pair(i, _):
            for half, (buf, sem) in enumerate(((ev0, se0), (ev1, se1))):
                ci = i * 2 + half
                pltpu.make_async_copy(
                    x_hbm.at[b, pl.ds(ci * CHW, CHW)], buf, sem).wait()

                def _g(j, acc):
                    for u in range(GUNROLL):
                        g = j * GUNROLL + u
                        m, pix, key = _ev_group(buf, ci, g)
                        plsc.store_scatter(band, [pix], key, mask=m)
                        back = plsc.load_gather(band, [pix], mask=m)
                        acc = acc | (m & (back < key))
                    return acc
                acc = lax.fori_loop(0, GROUPS // GUNROLL, _g, falses)

                def _fixpass(_acc):
                    def _fg(g, a):
                        m, pix, key = _ev_group(buf, ci, g)
                        back = plsc.load_gather(band, [pix], mask=m)
                        n = m & (back < key)
                        plsc.store_scatter(band, [pix], key, mask=n)
                        back2 = plsc.load_gather(band, [pix], mask=n)
                        return a | (n & (back2 < key))
                    return lax.fori_loop(0, GROUPS, _fg, falses)
                lax.while_loop(lambda a: jnp.any(a), _fixpass, acc)

                @pl.when(ci + 2 < NCHUNK)
                def _():
                    pltpu.async_copy(
                        x_hbm.at[b, pl.ds((ci + 2) * CHW, CHW)], buf, sem)
            return 0
        lax.fori_loop(0, NCHUNK // 2, _pair, 0)

        obufs = (ob0, ob1)
        osems = (so0, so1)

        def _off(oc):
            return (row_lo + oc * OROWS) * (W * 3)

        for oc in range(OCHUNK):
            buf, sem = obufs[oc % 2], osems[oc % 2]
            if oc >= 2:
                pltpu.make_async_copy(
                    buf, out_hbm.at[b, pl.ds(_off(oc - 2), OWORDS)], sem).wait()

            def _pg(v, _, oc=oc, buf=buf):
                for u in range(OUNROLL):
                    vg = v * OUNROLL + u
                    pbase = oc * (OROWS * W) + vg * L
                    key = band[pl.ds(pbase, L)]
                    band[pl.ds(pbase, L)] = neg1
                    touched = key >= 0
                    polb = (key & 1) == 1
                    y0 = jnp.where(touched & polb, 0.0, 255.0)
                    y0 = y0.astype(jnp.float32)
                    y1 = jnp.where(touched & (~polb), 0.0, 255.0)
                    y1 = y1.astype(jnp.float32)
                    oidx = lane3 + vg * (L * 3)
                    plsc.store_scatter(buf, [oidx], y0)
                    plsc.store_scatter(buf, [oidx + 1], y1)
                    plsc.store_scatter(buf, [oidx + 2], y0 + y1)
                return 0
            lax.fori_loop(0, OGROUPS // OUNROLL, _pg, 0)
            pltpu.async_copy(buf, out_hbm.at[b, pl.ds(_off(oc), OWORDS)], sem)

        for oc in (OCHUNK - 2, OCHUNK - 1):
            pltpu.make_async_copy(
                obufs[oc % 2], out_hbm.at[b, pl.ds(_off(oc), OWORDS)],
                osems[oc % 2]).wait()
        return 0
    lax.fori_loop(0, B // NC, _batch, 0)


@jax.jit
def _e2img(xr):
    mesh = plsc.VectorSubcoreMesh(
        core_axis_name="c", subcore_axis_name="s", num_cores=NC, num_subcores=NS)
    f = pl.kernel(
        _sc_body,
        out_type=jax.ShapeDtypeStruct((B, H * W * 3), jnp.float32),
        mesh=mesh,
        scratch_types=[
            pltpu.VMEM((BAND * W,), jnp.int32),
            pltpu.VMEM((CHW,), jnp.float32),
            pltpu.VMEM((CHW,), jnp.float32),
            pltpu.VMEM((OWORDS,), jnp.float32),
            pltpu.VMEM((OWORDS,), jnp.float32),
            pltpu.SemaphoreType.DMA,
            pltpu.SemaphoreType.DMA,
            pltpu.SemaphoreType.DMA,
            pltpu.SemaphoreType.DMA,
        ],
        compiler_params=pltpu.CompilerParams(needs_layout_passes=False),
    )
    return f(xr)


def kernel(x):
    xr = x.reshape(B, N * 4)
    out = _e2img(xr)
    return out.reshape(B, H, W, 3)
